# Initial kernel scaffold; baseline (speedup 1.0000x reference)
import jax
import jax.numpy as jnp
from jax.experimental import pallas as pl


def kernel(inputs, vertex, face, full_nf_count, full_vt_map, filt_coeff, nv_in, dw1_0, W1_0, b1_0, dw2_0, W2_0, b2_0, dw1_1, W1_1, b1_1, dw2_1, W2_1, b2_1, dw1_2, W1_2, b1_2, dw2_2, W2_2, b2_2, dw1_3, W1_3, b1_3, dw2_3, W2_3, b2_3, W_t, b_t):
    raise NotImplementedError("write your pallas kernel here")


# speedup vs baseline: 1.1581x; 1.1581x over previous
import jax
import jax.numpy as jnp
from jax import lax
from jax.experimental import pallas as pl
from jax.experimental.pallas import tpu as pltpu
from jax.experimental.pallas import tpu_sc as plsc

NV = 100000
NF = 200000
K = 9
NF_PAD = 212992
NCHUNK = NF_PAD // 1024
FROWS = 3 * NF_PAD // 128
FROWS_A = 5120
MVR = FROWS_A // 32
NV_PAD = 115200

_mesh = plsc.VectorSubcoreMesh(core_axis_name="c", subcore_axis_name="s")


def _mv_body(face_hbm, vt_hbm, out_hbm, idx_v, out_v, sem):
    c = lax.axis_index("c")
    s = lax.axis_index("s")
    wid = s * 2 + c
    pltpu.sync_copy(face_hbm.at[pl.ds(wid * MVR, MVR)], idx_v)

    def grp(g, carry):
        r0 = g * 10
        descs = [pltpu.async_copy(vt_hbm.at[idx_v.at[r0 + j]],
                                  out_v.at[r0 + j], sem)
                 for j in range(10)]
        for d in descs:
            d.wait()
        return carry

    lax.fori_loop(0, MVR // 10, grp, 0)
    pltpu.sync_copy(out_v, out_hbm.at[pl.ds(wid * MVR, MVR)])


def _mv_precompute(ft2a, vt_map):
    return pl.kernel(
        _mv_body,
        out_type=jax.ShapeDtypeStruct((FROWS_A, 128), jnp.int32),
        mesh=_mesh,
        scratch_types=[
            pltpu.VMEM((MVR, 128), jnp.int32),
            pltpu.VMEM((MVR, 128), jnp.int32),
            pltpu.SemaphoreType.DMA,
        ],
    )(ft2a, vt_map)


def _make_gather_sc(spoff):
    def body(y_hbm, sp_hbm, ft_hbm, wf_hbm,
             fbuf, rs, sp0, sp1, sp2, sp3, sem):
        c = lax.axis_index("c")
        s = lax.axis_index("s")
        wid = s * 2 + c
        sps = [sp0, sp1, sp2, sp3]

        def tloop(t, carry):
            p = wid + 32 * t

            @pl.when(p < NCHUNK)
            def _():
                for k in range(3):
                    pltpu.sync_copy(
                        ft_hbm.at[pl.ds(k * (NF_PAD // 128) + p * 8, 8)],
                        fbuf.at[pl.ds(k * 8, 8)])
                for k in range(3):
                    off = k * NV
                    for q in range(8):
                        @plsc.parallel_loop(0, 128, 16, unroll=8)
                        def _(i, _k=k, _q=q, _off=off):
                            fbuf[_k * 8 + _q, pl.ds(i, 16)] = (
                                fbuf[_k * 8 + _q, pl.ds(i, 16)] + _off)

                def qloop(q, carry2):
                    fst = p * 1024 + q * 128
                    for b in range(4):
                        pltpu.sync_copy(
                            sp_hbm.at[pl.ds(spoff + b * NF_PAD + fst, 128)],
                            sps[b])

                    def fire(k, carry3):
                        pltpu.async_copy(
                            y_hbm.at[fbuf.at[k * 8 + q]],
                            rs.at[pl.ds(k * 128, 128)], sem)
                        return carry3

                    lax.fori_loop(0, 3, fire, 0)
                    for _ in range(3):
                        pltpu.make_async_copy(
                            y_hbm.at[pl.ds(0, 128)],
                            rs.at[pl.ds(0, 128)], sem).wait()
                    for b in range(4):
                        @plsc.parallel_loop(0, 128, 1, unroll=8)
                        def _(i, _b=b):
                            cs = pl.ds(_b * 16, 16)
                            t2 = (rs[i, cs] + rs[128 + i, cs]
                                  + rs[256 + i, cs])
                            sps[_b][i] = jnp.maximum(t2, 0.0) * sps[_b][i]
                    for b in range(4):
                        pltpu.sync_copy(
                            sps[b],
                            wf_hbm.at[pl.ds(b * NF_PAD + fst, 128)])
                    return carry2

                lax.fori_loop(0, 8, qloop, 0)
            return carry

        lax.fori_loop(0, 7, tloop, 0)

    return pl.kernel(
        body,
        out_type=jax.ShapeDtypeStruct((4 * NF_PAD, 16), jnp.float32),
        mesh=_mesh,
        scratch_types=[
            pltpu.VMEM((24, 128), jnp.int32),
            pltpu.VMEM((384, 128), jnp.float32),
            pltpu.VMEM((128, 16), jnp.float32),
            pltpu.VMEM((128, 16), jnp.float32),
            pltpu.VMEM((128, 16), jnp.float32),
            pltpu.VMEM((128, 16), jnp.float32),
            pltpu.SemaphoreType.DMA,
        ],
    )


def _y_call(parts, weights, bias, BN=1000):
    P = len(parts)
    dims = [int(p.shape[1]) for p in parts]

    def body(*refs):
        xs = refs[:P]
        ws = refs[P:2 * P]
        bia = refs[2 * P]
        out = refs[2 * P + 1]
        a = jnp.dot(xs[0][...], ws[0][0], preferred_element_type=jnp.float32)
        for t in range(1, P):
            a = a + jnp.dot(xs[t][...], ws[t][0],
                            preferred_element_type=jnp.float32)
        out[...] = a + bia[0]

    in_specs = (
        [pl.BlockSpec((BN, d), lambda i, m: (i, 0)) for d in dims]
        + [pl.BlockSpec((1, d, 128), lambda i, m: (m, 0, 0)) for d in dims]
        + [pl.BlockSpec((1, 1, 128), lambda i, m: (m, 0, 0))]
    )
    return pl.pallas_call(
        body,
        grid=(NV // BN, 3),
        in_specs=in_specs,
        out_specs=pl.BlockSpec((BN, 128), lambda i, m: (m * (NV // BN) + i, 0)),
        out_shape=jax.ShapeDtypeStruct((3 * NV, 128), jnp.float32),
    )(*parts, *weights, bias)


def _sp_call(filt_pad, dw2s, BF=2048):
    def body(x_ref, w_ref, out_ref):
        out_ref[...] = jnp.dot(x_ref[...], w_ref[0],
                               preferred_element_type=jnp.float32)

    return pl.pallas_call(
        body,
        grid=(NF_PAD // BF, 16),
        in_specs=[pl.BlockSpec((BF, 16), lambda i, m: (i, 0)),
                  pl.BlockSpec((1, 16, 16), lambda i, m: (m, 0, 0))],
        out_specs=pl.BlockSpec((BF, 16),
                               lambda i, m: (m * (NF_PAD // BF) + i, 0)),
        out_shape=jax.ShapeDtypeStruct((16 * NF_PAD, 16), jnp.float32),
    )(filt_pad, dw2s)


def _post_call(acc_flat, ndf, W2, b2, BN=800):
    c2 = int(W2.shape[1])
    nvb = NV_PAD // BN

    def body(a0, a1, a2, a3, nd_ref, w_ref, b_ref, out_ref):
        cat = jnp.concatenate([a0[...], a1[...], a2[...], a3[...]], axis=1)
        inv = 1.0 / jnp.maximum(nd_ref[...], 1.0)
        h = jnp.dot(cat * inv, w_ref[...], preferred_element_type=jnp.float32)
        out_ref[...] = jnp.maximum(h + b_ref[...], 0.0)

    in_specs = [pl.BlockSpec((BN, 16), (lambda i, _q=q: (_q * nvb + i, 0)))
                for q in range(4)]
    in_specs += [pl.BlockSpec((BN, 1), lambda i: (i, 0)),
                 pl.BlockSpec((64, c2), lambda i: (0, 0)),
                 pl.BlockSpec((1, c2), lambda i: (0, 0))]
    return pl.pallas_call(
        body,
        grid=(NV // BN,),
        in_specs=in_specs,
        out_specs=pl.BlockSpec((BN, c2), lambda i: (i, 0)),
        out_shape=jax.ShapeDtypeStruct((NV, c2), jnp.float32),
    )(acc_flat, acc_flat, acc_flat, acc_flat, ndf, W2, b2.reshape(1, c2))


def _final_call(parts, wparts, b_t, BN=1000):
    P = len(parts)
    dims = [int(p.shape[1]) for p in parts]

    def body(*refs):
        xs = refs[:P]
        ws = refs[P:2 * P]
        bia = refs[2 * P]
        out = refs[2 * P + 1]
        a = jnp.dot(xs[0][...], ws[0][...], preferred_element_type=jnp.float32)
        for t in range(1, P):
            a = a + jnp.dot(xs[t][...], ws[t][...],
                            preferred_element_type=jnp.float32)
        out[...] = jnp.maximum(a + bia[...], 0.0)

    in_specs = (
        [pl.BlockSpec((BN, d), lambda i: (i, 0)) for d in dims]
        + [pl.BlockSpec((d, 128), lambda i: (0, 0)) for d in dims]
        + [pl.BlockSpec((1, 128), lambda i: (0, 0))]
    )
    return pl.pallas_call(
        body,
        grid=(NV // BN,),
        in_specs=in_specs,
        out_specs=pl.BlockSpec((BN, 128), lambda i: (i, 0)),
        out_shape=jax.ShapeDtypeStruct((NV, 128), jnp.float32),
    )(*parts, *wparts, b_t.reshape(1, 128))


def _wy3(dw1, W1):
    w = jnp.stack([dw1[k][:, None] * W1 for k in range(3)])
    return jnp.pad(w, ((0, 0), (0, 0), (0, 128 - w.shape[2])))


def _by3(b1):
    b = jnp.pad(b1, (0, 128 - b1.shape[0])) / 3.0
    return jnp.broadcast_to(b, (3, 1, 128))


def kernel(inputs, vertex, face, full_nf_count, full_vt_map, filt_coeff, nv_in,
           dw1_0, W1_0, b1_0, dw2_0, W2_0, b2_0,
           dw1_1, W1_1, b1_1, dw2_1, W2_1, b2_1,
           dw1_2, W1_2, b1_2, dw2_2, W2_2, b2_2,
           dw1_3, W1_3, b1_3, dw2_3, W2_3, b2_3,
           W_t, b_t):
    face_flat = jnp.pad(face.T, ((0, 0), (0, NF_PAD - NF))).reshape(-1)
    ft2a = jnp.pad(face_flat, (0, FROWS_A * 128 - 3 * NF_PAD)).reshape(
        FROWS_A, 128)
    mt2 = _mv_precompute(ft2a, full_vt_map)
    filt_pad = jnp.pad(filt_coeff, ((0, NF_PAD - NF), (0, 16 - K)))
    dw2_all = [dw2_0, dw2_1, dw2_2, dw2_3]
    dw2s = jnp.stack([
        jnp.pad(dw2_all[m // 4], ((0, 16 - K), (0, 0)))[:, (m % 4) * 16:((m % 4) + 1) * 16]
        for m in range(16)])
    spflat = _sp_call(filt_pad, dw2s)
    ndf = full_nf_count.astype(jnp.float32).reshape(NV, 1)

    gather_calls = [_make_gather_sc(l * 4 * NF_PAD) for l in range(4)]
    mt_rows = mt2.reshape(-1)[:3 * NF_PAD].reshape(3, NF_PAD)

    def layer(l, parts, wparts, b1, W2, b2):
        y = _y_call(parts, wparts, _by3(b1))
        wf = gather_calls[l](y, spflat, ft2a)
        acc4 = []
        for b in range(4):
            a = jnp.zeros((NV_PAD, 16), jnp.float32)
            for k in range(3):
                a = a.at[mt_rows[k]].add(wf[b * NF_PAD:(b + 1) * NF_PAD])
            acc4.append(a)
        acc = jnp.concatenate(acc4, axis=0)
        return _post_call(acc, ndf, W2, b2)

    net1 = layer(0, [inputs], [_wy3(dw1_0, W1_0)], b1_0, W2_0, b2_0)
    net2 = layer(1, [net1], [_wy3(dw1_1, W1_1)], b1_1, W2_1, b2_1)
    wy2 = _wy3(dw1_2, W1_2)
    net3 = layer(2, [inputs, net2], [wy2[:, :128], wy2[:, 128:]],
                 b1_2, W2_2, b2_2)
    net4 = layer(3, [net3], [_wy3(dw1_3, W1_3)], b1_3, W2_3, b2_3)
    return _final_call([inputs, net2, net4],
                       [W_t[:128], W_t[128:144], W_t[144:160]], b_t)
